# baseline (speedup 1.0000x reference)
import functools

import jax
import jax.numpy as jnp
from jax.experimental import pallas as pl
from jax.experimental.pallas import tpu as pltpu

N = 4096
IN_FEAT = 128
BLK = 256
GRID = N // BLK
NIN = 4
NOUT = 3


def _dot(a, b):
    return jax.lax.dot_general(
        a, b, (((1,), (0,)), ((), ())),
        preferred_element_type=jnp.float32,
    )


def _kernel(adj_hbm, b_ref, w_enc_ref, x_latent_hbm, x_recon_ref,
            adj_buf, xl_buf, ab_acc, in_sems, out_sems):
    def copy_in(i):
        return pltpu.make_async_copy(
            adj_hbm.at[pl.ds(i * BLK, BLK), :],
            adj_buf.at[i % NIN],
            in_sems.at[i % NIN],
        )

    def copy_out(i):
        return pltpu.make_async_copy(
            xl_buf.at[i % NOUT],
            x_latent_hbm.at[pl.ds(i * BLK, BLK), :],
            out_sems.at[i % NOUT],
        )

    for j in range(NIN - 1):
        copy_in(j).start()
    for i in range(GRID):
        if i + NIN - 1 < GRID:
            copy_in(i + NIN - 1).start()
        copy_in(i).wait()
        ab = _dot(adj_buf[i % NIN], b_ref[...])
        ab_acc[pl.ds(i * BLK, BLK), :] = ab
        if i >= NOUT:
            copy_out(i - NOUT).wait()
        xl_buf[i % NOUT] = _dot(ab[:, :IN_FEAT], w_enc_ref[...])
        copy_out(i).start()

    a = ab_acc[:, :IN_FEAT]
    y = ab_acc[:, IN_FEAT:]
    m = _dot(w_enc_ref[...].astype(jnp.float32), y)
    x_recon_ref[...] = _dot(a, m)
    for j in range(NOUT):
        copy_out(GRID - NOUT + j).wait()


@jax.jit
def _run(feat, adj, W_enc, W_dec):
    b = jnp.concatenate([feat, W_dec], axis=1).astype(jnp.bfloat16)
    x_latent, x_recon = pl.pallas_call(
        _kernel,
        in_specs=[
            pl.BlockSpec(memory_space=pltpu.MemorySpace.HBM),
            pl.BlockSpec((N, 2 * IN_FEAT), lambda: (0, 0)),
            pl.BlockSpec((IN_FEAT, N), lambda: (0, 0)),
        ],
        out_specs=[
            pl.BlockSpec(memory_space=pltpu.MemorySpace.HBM),
            pl.BlockSpec((N, IN_FEAT), lambda: (0, 0)),
        ],
        out_shape=[
            jax.ShapeDtypeStruct((N, N), jnp.float32),
            jax.ShapeDtypeStruct((N, IN_FEAT), jnp.float32),
        ],
        scratch_shapes=[
            pltpu.VMEM((NIN, BLK, N), jnp.float32),
            pltpu.VMEM((NOUT, BLK, N), jnp.float32),
            pltpu.VMEM((N, 2 * IN_FEAT), jnp.float32),
            pltpu.SemaphoreType.DMA((NIN,)),
            pltpu.SemaphoreType.DMA((NOUT,)),
        ],
    )(adj, b, W_enc.astype(jnp.bfloat16))
    return x_latent, x_recon


def kernel(feat, adj, W_enc, W_dec):
    return _run(feat, adj, W_enc, W_dec)


# speedup vs baseline: 1.0059x; 1.0059x over previous
import functools

import jax
import jax.numpy as jnp
from jax.experimental import pallas as pl
from jax.experimental.pallas import tpu as pltpu

N = 4096
IN_FEAT = 128
BLK = 512
GRID = N // BLK
NIN = 3
NOUT = 2


def _dot(a, b):
    return jax.lax.dot_general(
        a, b, (((1,), (0,)), ((), ())),
        preferred_element_type=jnp.float32,
    )


def _kernel(adj_hbm, b_ref, w_enc_ref, x_latent_hbm, x_recon_ref,
            adj_buf, xl_buf, ab_acc, in_sems, out_sems):
    def copy_in(i):
        return pltpu.make_async_copy(
            adj_hbm.at[pl.ds(i * BLK, BLK), :],
            adj_buf.at[i % NIN],
            in_sems.at[i % NIN],
        )

    def copy_out(i):
        return pltpu.make_async_copy(
            xl_buf.at[i % NOUT],
            x_latent_hbm.at[pl.ds(i * BLK, BLK), :],
            out_sems.at[i % NOUT],
        )

    for j in range(NIN - 1):
        copy_in(j).start()
    for i in range(GRID):
        if i + NIN - 1 < GRID:
            copy_in(i + NIN - 1).start()
        copy_in(i).wait()
        ab = _dot(adj_buf[i % NIN], b_ref[...])
        ab_acc[pl.ds(i * BLK, BLK), :] = ab
        if i >= NOUT:
            copy_out(i - NOUT).wait()
        xl_buf[i % NOUT] = _dot(ab[:, :IN_FEAT], w_enc_ref[...])
        copy_out(i).start()

    a = ab_acc[:, :IN_FEAT]
    y = ab_acc[:, IN_FEAT:]
    m = _dot(w_enc_ref[...].astype(jnp.float32), y)
    x_recon_ref[...] = _dot(a, m)
    for j in range(NOUT):
        copy_out(GRID - NOUT + j).wait()


@jax.jit
def _run(feat, adj, W_enc, W_dec):
    b = jnp.concatenate([feat, W_dec], axis=1).astype(jnp.bfloat16)
    x_latent, x_recon = pl.pallas_call(
        _kernel,
        in_specs=[
            pl.BlockSpec(memory_space=pltpu.MemorySpace.HBM),
            pl.BlockSpec((N, 2 * IN_FEAT), lambda: (0, 0)),
            pl.BlockSpec((IN_FEAT, N), lambda: (0, 0)),
        ],
        out_specs=[
            pl.BlockSpec(memory_space=pltpu.MemorySpace.HBM),
            pl.BlockSpec((N, IN_FEAT), lambda: (0, 0)),
        ],
        out_shape=[
            jax.ShapeDtypeStruct((N, N), jnp.float32),
            jax.ShapeDtypeStruct((N, IN_FEAT), jnp.float32),
        ],
        scratch_shapes=[
            pltpu.VMEM((NIN, BLK, N), jnp.float32),
            pltpu.VMEM((NOUT, BLK, N), jnp.float32),
            pltpu.VMEM((N, 2 * IN_FEAT), jnp.float32),
            pltpu.SemaphoreType.DMA((NIN,)),
            pltpu.SemaphoreType.DMA((NOUT,)),
        ],
    )(adj, b, W_enc.astype(jnp.bfloat16))
    return x_latent, x_recon


def kernel(feat, adj, W_enc, W_dec):
    return _run(feat, adj, W_enc, W_dec)
